# Initial kernel scaffold; baseline (speedup 1.0000x reference)
import jax
import jax.numpy as jnp
from jax.experimental import pallas as pl


def kernel(x, gate_w, Wg, Wu, Wd):
    raise NotImplementedError("write your pallas kernel here")


# speedup vs baseline: 1.0970x; 1.0970x over previous
import functools

import jax
import jax.numpy as jnp
from jax.experimental import pallas as pl
from jax.experimental.pallas import tpu as pltpu

D = 768
I = 384
E = 16
N = 2048


def _moe_body(x_ref, gw_ref, wg_ref, wu_ref, wd_ref, out_ref, comb_ref):
    e = pl.program_id(0)

    @pl.when(e == 0)
    def _router():
        xt = x_ref[...]
        logits = jax.lax.dot_general(
            xt, gw_ref[...],
            dimension_numbers=(((1,), (1,)), ((), ())),
            preferred_element_type=jnp.float32,
        )
        m = jnp.max(logits, axis=1, keepdims=True)
        ex = jnp.exp(logits - m)
        scores = ex / jnp.sum(ex, axis=1, keepdims=True)
        lane = jax.lax.broadcasted_iota(jnp.int32, (N, E), 1)
        m1 = jnp.max(scores, axis=1, keepdims=True)
        a1 = jnp.min(jnp.where(scores == m1, lane, E), axis=1, keepdims=True)
        masked = jnp.where(lane == a1, -jnp.inf, scores)
        m2 = jnp.max(masked, axis=1, keepdims=True)
        a2 = jnp.min(jnp.where(masked == m2, lane, E), axis=1, keepdims=True)
        comb_ref[...] = (
            jnp.where(lane == a1, m1, 0.0) + jnp.where(lane == a2, m2, 0.0)
        )
        out_ref[...] = jnp.zeros_like(out_ref)

    lane = jax.lax.broadcasted_iota(jnp.int32, (N, E), 1)
    w_e = jnp.sum(
        jnp.where(lane == e, comb_ref[...], 0.0), axis=1, keepdims=True
    )

    xt = x_ref[...]
    h = jnp.dot(xt, wg_ref[0], preferred_element_type=jnp.float32)
    u = jnp.dot(xt, wu_ref[0], preferred_element_type=jnp.float32)
    a = (h * jax.lax.logistic(h)) * u
    y = jnp.dot(a, wd_ref[0], preferred_element_type=jnp.float32)
    out_ref[...] += w_e * y


@jax.jit
def kernel(x, gate_w, Wg, Wu, Wd):
    b, s, d = x.shape
    xt = x.reshape(-1, d)
    WgT = Wg.transpose(0, 2, 1)
    WuT = Wu.transpose(0, 2, 1)
    WdT = Wd.transpose(0, 2, 1)

    out = pl.pallas_call(
        _moe_body,
        grid=(E,),
        in_specs=[
            pl.BlockSpec((N, D), lambda e: (0, 0)),
            pl.BlockSpec((E, D), lambda e: (0, 0)),
            pl.BlockSpec((1, D, I), lambda e: (e, 0, 0)),
            pl.BlockSpec((1, D, I), lambda e: (e, 0, 0)),
            pl.BlockSpec((1, I, D), lambda e: (e, 0, 0)),
        ],
        out_specs=pl.BlockSpec((N, D), lambda e: (0, 0)),
        out_shape=jax.ShapeDtypeStruct((N, D), jnp.float32),
        scratch_shapes=[pltpu.VMEM((N, E), jnp.float32)],
    )(xt, gate_w, WgT, WuT, WdT)
    return out.reshape(b, s, d)


# speedup vs baseline: 1.1769x; 1.0728x over previous
import functools

import jax
import jax.numpy as jnp
from jax import lax
from jax.experimental import pallas as pl
from jax.experimental.pallas import tpu as pltpu
from jax.experimental.pallas import tpu_sc as plsc

D = 768
I = 384
E = 16
N = 2048
K = 2
NK = N * K
R = 256
T = NK // R
G = T + E
NC = 2
NS = 16
NW = NC * NS
CHUNK = N // NW
LANES = 16


def _router_body(x_ref, gw_ref, route_ref, meta_ref):
    xt = x_ref[...]
    logits = lax.dot_general(
        xt, gw_ref[...],
        dimension_numbers=(((1,), (1,)), ((), ())),
        preferred_element_type=jnp.float32,
    )
    m = jnp.max(logits, axis=1, keepdims=True)
    ex = jnp.exp(logits - m)
    scores = ex / jnp.sum(ex, axis=1, keepdims=True)
    lane = lax.broadcasted_iota(jnp.int32, (N, E), 1)
    m1 = jnp.max(scores, axis=1, keepdims=True)
    a1 = jnp.min(jnp.where(scores == m1, lane, E), axis=1, keepdims=True)
    masked = jnp.where(lane == a1, -jnp.inf, scores)
    m2 = jnp.max(masked, axis=1, keepdims=True)
    a2 = jnp.min(jnp.where(masked == m2, lane, E), axis=1, keepdims=True)

    oh1 = (lane == a1).astype(jnp.float32)
    oh2 = (lane == a2).astype(jnp.float32)
    hist = oh1 + oh2

    c = hist
    step = 1
    while step < N:
        c = c + jnp.concatenate(
            [jnp.zeros((step, E), jnp.float32), c[: N - step]], axis=0
        )
        step *= 2
    base = c - hist
    totals = c[N - 1 : N, :]

    o = totals
    for sh in (1, 2, 4, 8):
        o = o + jnp.concatenate(
            [jnp.zeros((1, sh), jnp.float32), o[:, : E - sh]], axis=1
        )
    offs = o - totals

    slotpos = offs + base
    slot0 = jnp.sum(oh1 * slotpos, axis=1, keepdims=True)
    slot1 = jnp.sum(oh2 * slotpos, axis=1, keepdims=True)

    lane128 = lax.broadcasted_iota(jnp.int32, (N, 128), 1)
    route_ref[...] = (
        jnp.where(lane128 == 0, slot0, 0.0)
        + jnp.where(lane128 == 1, slot1, 0.0)
        + jnp.where(lane128 == 2, m1, 0.0)
        + jnp.where(lane128 == 3, m2, 0.0)
    )

    ends = offs + totals
    tt = lax.broadcasted_iota(jnp.int32, (T, E), 0).astype(jnp.float32)
    inter = (
        (offs < (tt + 1.0) * R) & (ends > tt * R) & (totals > 0)
    ).astype(jnp.float32)

    colcum = inter
    for sh in (1, 2, 4, 8):
        colcum = colcum + jnp.concatenate(
            [jnp.zeros((T, sh), jnp.float32), colcum[:, : E - sh]], axis=1
        )
    rowtot = colcum[:, E - 1 : E]
    rowbase = rowtot
    for sh in (1, 2, 4, 8):
        rowbase = rowbase + jnp.concatenate(
            [jnp.zeros((sh, 1), jnp.float32), rowbase[: T - sh]], axis=0
        )
    rowbase = rowbase - rowtot
    rank = rowbase + colcum - inter

    g_lane = lax.broadcasted_iota(jnp.int32, (T, E, 128), 2).astype(jnp.float32)
    sel = ((rank[:, :, None] == g_lane) & (inter[:, :, None] > 0)).astype(
        jnp.float32
    )
    t3 = lax.broadcasted_iota(jnp.int32, (T, E, 128), 0).astype(jnp.float32)
    e3 = lax.broadcasted_iota(jnp.int32, (T, E, 128), 1).astype(jnp.float32)
    map_t = jnp.sum(jnp.sum(sel * t3, axis=0), axis=0)[None, :]
    map_e = jnp.sum(jnp.sum(sel * e3, axis=0), axis=0)[None, :]
    vld = jnp.sum(jnp.sum(sel, axis=0), axis=0)[None, :]
    map_t = map_t + (1.0 - vld) * float(T - 1)
    map_e = map_e + (1.0 - vld) * float(E - 1)

    lane1 = lax.broadcasted_iota(jnp.int32, (1, 128), 1)
    offs_pad = jnp.concatenate(
        [offs, jnp.zeros((1, 128 - E), jnp.float32)], axis=1
    )
    offs17 = offs_pad + jnp.where(lane1 == E, float(NK), 0.0)

    row8 = lax.broadcasted_iota(jnp.int32, (8, 128), 0)
    meta_ref[...] = (
        jnp.where(row8 == 0, offs17, 0.0)
        + jnp.where(row8 == 1, map_t, 0.0)
        + jnp.where(row8 == 2, map_e, 0.0)
        + jnp.where(row8 == 3, vld, 0.0)
    )


def _router(xt, gate_w):
    return pl.pallas_call(
        _router_body,
        in_specs=[
            pl.BlockSpec((N, D), lambda: (0, 0)),
            pl.BlockSpec((E, D), lambda: (0, 0)),
        ],
        out_specs=[
            pl.BlockSpec((N, 128), lambda: (0, 0)),
            pl.BlockSpec((8, 128), lambda: (0, 0)),
        ],
        out_shape=[
            jax.ShapeDtypeStruct((N, 128), jnp.float32),
            jax.ShapeDtypeStruct((8, 128), jnp.float32),
        ],
    )(xt, gate_w)


def _dispatch(xt, slot0, slot1):
    mesh = plsc.VectorSubcoreMesh(core_axis_name="c", subcore_axis_name="s")

    @functools.partial(
        pl.kernel,
        mesh=mesh,
        out_type=jax.ShapeDtypeStruct((NK, D), jnp.float32),
        scratch_types=[
            pltpu.VMEM((CHUNK,), jnp.int32),
            pltpu.VMEM((CHUNK,), jnp.int32),
            pltpu.VMEM((CHUNK, D), jnp.float32),
            pltpu.SemaphoreType.DMA,
        ],
    )
    def k(x_hbm, s0_hbm, s1_hbm, xs_hbm, idx0_v, idx1_v, rows_v, sem):
        wid = lax.axis_index("s") * NC + lax.axis_index("c")
        b = wid * CHUNK
        pltpu.sync_copy(s0_hbm.at[pl.ds(b, CHUNK)], idx0_v)
        pltpu.sync_copy(s1_hbm.at[pl.ds(b, CHUNK)], idx1_v)
        pltpu.sync_copy(x_hbm.at[pl.ds(b, CHUNK)], rows_v)
        c0 = pltpu.async_copy(rows_v, xs_hbm.at[idx0_v], sem)
        c1 = pltpu.async_copy(rows_v, xs_hbm.at[idx1_v], sem)
        c0.wait()
        c1.wait()

    return k(xt, slot0, slot1)


def _ffn_body(mt_ref, me_ref, vd_ref, offs_ref,
              xs_ref, wg_ref, wu_ref, wd_ref, out_ref):
    g = pl.program_id(0)
    t = mt_ref[g]
    e = me_ref[g]
    v = vd_ref[g]
    lo = jnp.clip(offs_ref[e] - t * R, 0, R)
    hi = jnp.clip(offs_ref[e + 1] - t * R, 0, R)

    xt = xs_ref[...]
    h = jnp.dot(xt, wg_ref[0], preferred_element_type=jnp.float32)
    u = jnp.dot(xt, wu_ref[0], preferred_element_type=jnp.float32)
    a = (h * lax.logistic(h)) * u
    y = jnp.dot(a, wd_ref[0], preferred_element_type=jnp.float32)

    row = lax.broadcasted_iota(jnp.int32, (R, 1), 0)
    mask = (row >= lo) & (row < hi) & (v > 0)
    yw = jnp.where(mask, y, 0.0)

    prev = mt_ref[jnp.maximum(g - 1, 0)]
    first = (g == 0) | (t != prev)

    @pl.when(first)
    def _init():
        out_ref[...] = yw

    @pl.when(jnp.logical_not(first))
    def _accum():
        out_ref[...] += yw


def _ffn(map_t, map_e, vld, offs17, xs, WgT, WuT, WdT):
    grid_spec = pltpu.PrefetchScalarGridSpec(
        num_scalar_prefetch=4,
        grid=(G,),
        in_specs=[
            pl.BlockSpec((R, D), lambda g, mt, me, vd, of: (mt[g], 0)),
            pl.BlockSpec((1, D, I), lambda g, mt, me, vd, of: (me[g], 0, 0)),
            pl.BlockSpec((1, D, I), lambda g, mt, me, vd, of: (me[g], 0, 0)),
            pl.BlockSpec((1, I, D), lambda g, mt, me, vd, of: (me[g], 0, 0)),
        ],
        out_specs=pl.BlockSpec((R, D), lambda g, mt, me, vd, of: (mt[g], 0)),
    )
    return pl.pallas_call(
        _ffn_body,
        grid_spec=grid_spec,
        out_shape=jax.ShapeDtypeStruct((NK, D), jnp.float32),
    )(map_t, map_e, vld, offs17, xs, WgT, WuT, WdT)


def _combine(ys, slot0, slot1, w0, w1):
    mesh = plsc.VectorSubcoreMesh(core_axis_name="c", subcore_axis_name="s")

    @functools.partial(
        pl.kernel,
        mesh=mesh,
        out_type=jax.ShapeDtypeStruct((N, D), jnp.float32),
        scratch_types=[
            pltpu.VMEM((CHUNK,), jnp.int32),
            pltpu.VMEM((CHUNK,), jnp.int32),
            pltpu.VMEM((CHUNK + LANES,), jnp.float32),
            pltpu.VMEM((CHUNK + LANES,), jnp.float32),
            pltpu.VMEM((CHUNK, D), jnp.float32),
            pltpu.VMEM((CHUNK, D), jnp.float32),
            pltpu.SemaphoreType.DMA,
        ],
    )
    def k(ys_hbm, s0_hbm, s1_hbm, w0_hbm, w1_hbm, out_hbm,
          idx0_v, idx1_v, w0_v, w1_v, y0_v, y1_v, sem):
        wid = lax.axis_index("s") * NC + lax.axis_index("c")
        b = wid * CHUNK
        pltpu.sync_copy(s0_hbm.at[pl.ds(b, CHUNK)], idx0_v)
        pltpu.sync_copy(s1_hbm.at[pl.ds(b, CHUNK)], idx1_v)
        pltpu.sync_copy(w0_hbm.at[pl.ds(b, CHUNK)], w0_v.at[pl.ds(0, CHUNK)])
        pltpu.sync_copy(w1_hbm.at[pl.ds(b, CHUNK)], w1_v.at[pl.ds(0, CHUNK)])
        c0 = pltpu.async_copy(ys_hbm.at[idx0_v], y0_v, sem)
        c1 = pltpu.async_copy(ys_hbm.at[idx1_v], y1_v, sem)
        c0.wait()
        c1.wait()

        def body(r, carry):
            wv0 = jnp.full((LANES,), w0_v[pl.ds(r, LANES)][0], jnp.float32)
            wv1 = jnp.full((LANES,), w1_v[pl.ds(r, LANES)][0], jnp.float32)
            for j in range(D // LANES):
                sl = pl.ds(j * LANES, LANES)
                y0_v[r, sl] = wv0 * y0_v[r, sl] + wv1 * y1_v[r, sl]
            return carry

        lax.fori_loop(0, CHUNK, body, 0)
        pltpu.sync_copy(y0_v, out_hbm.at[pl.ds(b, CHUNK)])

    return k(ys, slot0, slot1, w0, w1)


@jax.jit
def kernel(x, gate_w, Wg, Wu, Wd):
    b, s, d = x.shape
    xt = x.reshape(-1, d)
    WgT = Wg.transpose(0, 2, 1)
    WuT = Wu.transpose(0, 2, 1)
    WdT = Wd.transpose(0, 2, 1)

    route, meta = _router(xt, gate_w)
    slot0 = route[:, 0].astype(jnp.int32)
    slot1 = route[:, 1].astype(jnp.int32)
    w0 = route[:, 2]
    w1 = route[:, 3]
    offs17 = meta[0, : E + 1].astype(jnp.int32)
    map_t = meta[1, :G].astype(jnp.int32)
    map_e = meta[2, :G].astype(jnp.int32)
    vld = meta[3, :G].astype(jnp.int32)

    xs = _dispatch(xt, slot0, slot1)
    ys = _ffn(map_t, map_e, vld, offs17, xs, WgT, WuT, WdT)
    out = _combine(ys, slot0, slot1, w0, w1)
    return out.reshape(b, s, d)


# speedup vs baseline: 1.7688x; 1.5030x over previous
import functools

import jax
import jax.numpy as jnp
from jax import lax
from jax.experimental import pallas as pl
from jax.experimental.pallas import tpu as pltpu
from jax.experimental.pallas import tpu_sc as plsc

D = 768
I = 384
E = 16
N = 2048
K = 2
NK = N * K
R = 256
T = NK // R
G = T + E
NC = 2
NS = 16
NW = NC * NS
CHUNK = N // NW
LANES = 16


def _router_body(x_ref, gw_ref, route_ref, meta_ref):
    xt = x_ref[...]
    logits = lax.dot_general(
        xt, gw_ref[...],
        dimension_numbers=(((1,), (1,)), ((), ())),
        preferred_element_type=jnp.float32,
    )
    m = jnp.max(logits, axis=1, keepdims=True)
    ex = jnp.exp(logits - m)
    scores = ex / jnp.sum(ex, axis=1, keepdims=True)
    lane = lax.broadcasted_iota(jnp.int32, (N, E), 1)
    m1 = jnp.max(scores, axis=1, keepdims=True)
    a1 = jnp.min(jnp.where(scores == m1, lane, E), axis=1, keepdims=True)
    masked = jnp.where(lane == a1, -jnp.inf, scores)
    m2 = jnp.max(masked, axis=1, keepdims=True)
    a2 = jnp.min(jnp.where(masked == m2, lane, E), axis=1, keepdims=True)

    oh1 = (lane == a1).astype(jnp.float32)
    oh2 = (lane == a2).astype(jnp.float32)
    hist = oh1 + oh2

    c = hist
    step = 1
    while step < N:
        c = c + jnp.concatenate(
            [jnp.zeros((step, E), jnp.float32), c[: N - step]], axis=0
        )
        step *= 2
    base = c - hist
    totals = c[N - 1 : N, :]

    o = totals
    for sh in (1, 2, 4, 8):
        o = o + jnp.concatenate(
            [jnp.zeros((1, sh), jnp.float32), o[:, : E - sh]], axis=1
        )
    offs = o - totals

    slotpos = offs + base
    slot0 = jnp.sum(oh1 * slotpos, axis=1, keepdims=True)
    slot1 = jnp.sum(oh2 * slotpos, axis=1, keepdims=True)

    lane128 = lax.broadcasted_iota(jnp.int32, (N, 128), 1)
    route_ref[...] = (
        jnp.where(lane128 == 0, slot0, 0.0)
        + jnp.where(lane128 == 1, slot1, 0.0)
        + jnp.where(lane128 == 2, m1, 0.0)
        + jnp.where(lane128 == 3, m2, 0.0)
    )

    ends = offs + totals
    tt = lax.broadcasted_iota(jnp.int32, (T, E), 0).astype(jnp.float32)
    inter = (
        (offs < (tt + 1.0) * R) & (ends > tt * R) & (totals > 0)
    ).astype(jnp.float32)

    colcum = inter
    for sh in (1, 2, 4, 8):
        colcum = colcum + jnp.concatenate(
            [jnp.zeros((T, sh), jnp.float32), colcum[:, : E - sh]], axis=1
        )
    rowtot = colcum[:, E - 1 : E]
    rowbase = rowtot
    for sh in (1, 2, 4, 8):
        rowbase = rowbase + jnp.concatenate(
            [jnp.zeros((sh, 1), jnp.float32), rowbase[: T - sh]], axis=0
        )
    rowbase = rowbase - rowtot
    rank = rowbase + colcum - inter

    g_lane = lax.broadcasted_iota(jnp.int32, (T, E, 128), 2).astype(jnp.float32)
    sel = ((rank[:, :, None] == g_lane) & (inter[:, :, None] > 0)).astype(
        jnp.float32
    )
    t3 = lax.broadcasted_iota(jnp.int32, (T, E, 128), 0).astype(jnp.float32)
    e3 = lax.broadcasted_iota(jnp.int32, (T, E, 128), 1).astype(jnp.float32)
    map_t = jnp.sum(jnp.sum(sel * t3, axis=0), axis=0)[None, :]
    map_e = jnp.sum(jnp.sum(sel * e3, axis=0), axis=0)[None, :]
    vld = jnp.sum(jnp.sum(sel, axis=0), axis=0)[None, :]
    map_t = map_t + (1.0 - vld) * float(T - 1)
    map_e = map_e + (1.0 - vld) * float(E - 1)

    lane1 = lax.broadcasted_iota(jnp.int32, (1, 128), 1)
    offs_pad = jnp.concatenate(
        [offs, jnp.zeros((1, 128 - E), jnp.float32)], axis=1
    )
    offs17 = offs_pad + jnp.where(lane1 == E, float(NK), 0.0)

    row8 = lax.broadcasted_iota(jnp.int32, (8, 128), 0)
    meta_ref[...] = (
        jnp.where(row8 == 0, offs17, 0.0)
        + jnp.where(row8 == 1, map_t, 0.0)
        + jnp.where(row8 == 2, map_e, 0.0)
        + jnp.where(row8 == 3, vld, 0.0)
    )


def _router(xt, gate_w):
    return pl.pallas_call(
        _router_body,
        in_specs=[
            pl.BlockSpec((N, D), lambda: (0, 0)),
            pl.BlockSpec((E, D), lambda: (0, 0)),
        ],
        out_specs=[
            pl.BlockSpec((N, 128), lambda: (0, 0)),
            pl.BlockSpec((8, 128), lambda: (0, 0)),
        ],
        out_shape=[
            jax.ShapeDtypeStruct((N, 128), jnp.float32),
            jax.ShapeDtypeStruct((8, 128), jnp.float32),
        ],
    )(xt, gate_w)


def _dispatch(xt, slot0, slot1):
    mesh = plsc.VectorSubcoreMesh(core_axis_name="c", subcore_axis_name="s")

    @functools.partial(
        pl.kernel,
        mesh=mesh,
        out_type=jax.ShapeDtypeStruct((NK, D), jnp.float32),
        scratch_types=[
            pltpu.VMEM((CHUNK,), jnp.int32),
            pltpu.VMEM((CHUNK,), jnp.int32),
            pltpu.VMEM((CHUNK, D), jnp.float32),
            pltpu.SemaphoreType.DMA,
        ],
    )
    def k(x_hbm, s0_hbm, s1_hbm, xs_hbm, idx0_v, idx1_v, rows_v, sem):
        wid = lax.axis_index("s") * NC + lax.axis_index("c")
        b = wid * CHUNK
        pltpu.sync_copy(s0_hbm.at[pl.ds(b, CHUNK)], idx0_v)
        pltpu.sync_copy(s1_hbm.at[pl.ds(b, CHUNK)], idx1_v)
        pltpu.sync_copy(x_hbm.at[pl.ds(b, CHUNK)], rows_v)
        c0 = pltpu.async_copy(rows_v, xs_hbm.at[idx0_v], sem)
        c1 = pltpu.async_copy(rows_v, xs_hbm.at[idx1_v], sem)
        c0.wait()
        c1.wait()

    return k(xt, slot0, slot1)


def _ffn_body(meta_ref, xs_ref, wg_ref, wu_ref, wd_ref, out_ref):
    g = pl.program_id(0)
    t = meta_ref[1, g]
    e = meta_ref[2, g]
    v = meta_ref[3, g]
    lo = jnp.clip(meta_ref[0, e] - t * R, 0, R)
    hi = jnp.clip(meta_ref[0, e + 1] - t * R, 0, R)

    xt = xs_ref[...]
    cdim = (((1,), (1,)), ((), ()))
    h = lax.dot_general(xt, wg_ref[0], cdim,
                        preferred_element_type=jnp.float32)
    u = lax.dot_general(xt, wu_ref[0], cdim,
                        preferred_element_type=jnp.float32)
    a = (h * lax.logistic(h)) * u
    y = lax.dot_general(a, wd_ref[0], cdim,
                        preferred_element_type=jnp.float32)

    row = lax.broadcasted_iota(jnp.int32, (R, 1), 0)
    mask = (row >= lo) & (row < hi) & (v > 0)
    yw = jnp.where(mask, y, 0.0)

    prev = meta_ref[1, jnp.maximum(g - 1, 0)]
    first = (g == 0) | (t != prev)

    @pl.when(first)
    def _init():
        out_ref[...] = yw

    @pl.when(jnp.logical_not(first))
    def _accum():
        out_ref[...] += yw


def _ffn(meta_i, xs, Wg, Wu, Wd):
    grid_spec = pltpu.PrefetchScalarGridSpec(
        num_scalar_prefetch=1,
        grid=(G,),
        in_specs=[
            pl.BlockSpec((R, D), lambda g, m: (m[1, g], 0)),
            pl.BlockSpec((1, I, D), lambda g, m: (m[2, g], 0, 0)),
            pl.BlockSpec((1, I, D), lambda g, m: (m[2, g], 0, 0)),
            pl.BlockSpec((1, D, I), lambda g, m: (m[2, g], 0, 0)),
        ],
        out_specs=pl.BlockSpec((R, D), lambda g, m: (m[1, g], 0)),
    )
    return pl.pallas_call(
        _ffn_body,
        grid_spec=grid_spec,
        out_shape=jax.ShapeDtypeStruct((NK, D), jnp.float32),
    )(meta_i, xs, Wg, Wu, Wd)


def _combine(ys, slot0, slot1, w0, w1):
    mesh = plsc.VectorSubcoreMesh(core_axis_name="c", subcore_axis_name="s")

    @functools.partial(
        pl.kernel,
        mesh=mesh,
        out_type=jax.ShapeDtypeStruct((N, D), jnp.float32),
        scratch_types=[
            pltpu.VMEM((CHUNK,), jnp.int32),
            pltpu.VMEM((CHUNK,), jnp.int32),
            pltpu.VMEM((CHUNK + LANES,), jnp.float32),
            pltpu.VMEM((CHUNK + LANES,), jnp.float32),
            pltpu.VMEM((CHUNK, D), jnp.float32),
            pltpu.VMEM((CHUNK, D), jnp.float32),
            pltpu.SemaphoreType.DMA,
        ],
    )
    def k(ys_hbm, s0_hbm, s1_hbm, w0_hbm, w1_hbm, out_hbm,
          idx0_v, idx1_v, w0_v, w1_v, y0_v, y1_v, sem):
        wid = lax.axis_index("s") * NC + lax.axis_index("c")
        b = wid * CHUNK
        pltpu.sync_copy(s0_hbm.at[pl.ds(b, CHUNK)], idx0_v)
        pltpu.sync_copy(s1_hbm.at[pl.ds(b, CHUNK)], idx1_v)
        pltpu.sync_copy(w0_hbm.at[pl.ds(b, CHUNK)], w0_v.at[pl.ds(0, CHUNK)])
        pltpu.sync_copy(w1_hbm.at[pl.ds(b, CHUNK)], w1_v.at[pl.ds(0, CHUNK)])
        c0 = pltpu.async_copy(ys_hbm.at[idx0_v], y0_v, sem)
        c1 = pltpu.async_copy(ys_hbm.at[idx1_v], y1_v, sem)
        c0.wait()
        c1.wait()

        def body(r, carry):
            wv0 = jnp.full((LANES,), w0_v[pl.ds(r, LANES)][0], jnp.float32)
            wv1 = jnp.full((LANES,), w1_v[pl.ds(r, LANES)][0], jnp.float32)
            for j in range(D // LANES):
                sl = pl.ds(j * LANES, LANES)
                y0_v[r, sl] = wv0 * y0_v[r, sl] + wv1 * y1_v[r, sl]
            return carry

        lax.fori_loop(0, CHUNK, body, 0)
        pltpu.sync_copy(y0_v, out_hbm.at[pl.ds(b, CHUNK)])

    return k(ys, slot0, slot1, w0, w1)


@jax.jit
def kernel(x, gate_w, Wg, Wu, Wd):
    b, s, d = x.shape
    xt = x.reshape(-1, d)

    route, meta = _router(xt, gate_w)
    slot0 = route[:, 0].astype(jnp.int32)
    slot1 = route[:, 1].astype(jnp.int32)
    w0 = route[:, 2]
    w1 = route[:, 3]
    meta_i = meta.astype(jnp.int32)

    xs = _dispatch(xt, slot0, slot1)
    ys = _ffn(meta_i, xs, Wg, Wu, Wd)
    out = _combine(ys, slot0, slot1, w0, w1)
    return out.reshape(b, s, d)
